# Initial kernel scaffold; baseline (speedup 1.0000x reference)
import jax
import jax.numpy as jnp
from jax.experimental import pallas as pl


def kernel(q, mu, W_ij, dir_ij, pairlist, W1, b1, W2, b2):
    raise NotImplementedError("write your pallas kernel here")


# speedup vs baseline: 6.9651x; 6.9651x over previous
import functools

import jax
import jax.numpy as jnp
from jax import lax
from jax.experimental import pallas as pl
from jax.experimental.pallas import tpu as pltpu
from jax.experimental.pallas import tpu_sc as plsc

NC = 2
NS = 16
L = 16
N = 10000
P = 320000
F = 128
F3 = 3 * F
N2 = N // 2
E = 80
CH = 40
NCHUNK = P // E
NCHUNK2 = NCHUNK // NC
EM = 40
NCHM = P // EM
NCHM2 = NCHM // NC


def _mlp_body(q_ref, w1_ref, b1_ref, w2_ref, b2_ref, xa_ref, xc_ref):
    h = jnp.dot(q_ref[...], w1_ref[...], preferred_element_type=jnp.float32)
    h = h + b1_ref[...]
    h = h * jax.nn.sigmoid(h)
    x = jnp.dot(h, w2_ref[...], preferred_element_type=jnp.float32)
    x = x + b2_ref[...]
    xa_ref[...] = x[:, 0:F]
    xc_ref[...] = x[:, F:F3]


def _mlp(q2, W1, b1r, W2, b2r):
    BR = 2000
    return pl.pallas_call(
        _mlp_body,
        grid=(N // BR,),
        in_specs=[
            pl.BlockSpec((BR, F), lambda i: (i, 0)),
            pl.BlockSpec((F, F), lambda i: (0, 0)),
            pl.BlockSpec((1, F), lambda i: (0, 0)),
            pl.BlockSpec((F, F3), lambda i: (0, 0)),
            pl.BlockSpec((1, F3), lambda i: (0, 0)),
        ],
        out_specs=[
            pl.BlockSpec((BR, F), lambda i: (i, 0)),
            pl.BlockSpec((BR, 2 * F), lambda i: (i, 0)),
        ],
        out_shape=[
            jax.ShapeDtypeStruct((N, F), jnp.float32),
            jax.ShapeDtypeStruct((N, 2 * F), jnp.float32),
        ],
    )(q2, W1, b1r, W2, b2r)


def _epilogue_body(q_ref, mu_ref, p0_ref, p1_ref,
                   a0_ref, a1_ref, a2_ref, b0_ref, b1_ref, b2_ref,
                   qo_ref, mo_ref):
    qo_ref[...] = q_ref[...] + p0_ref[...] + p1_ref[...]
    a_refs = (a0_ref, a1_ref, a2_ref)
    b_refs = (b0_ref, b1_ref, b2_ref)
    for d in range(3):
        sl = pl.ds(d * F, F)
        mo_ref[:, sl] = mu_ref[:, sl] + a_refs[d][...] + b_refs[d][...]


def _epilogue(q2, mu_f, p0, p1, dmu_parts):
    BR = 2000
    nb = N // BR
    return pl.pallas_call(
        _epilogue_body,
        grid=(nb,),
        in_specs=[
            pl.BlockSpec((BR, F), lambda i: (i, 0)),
            pl.BlockSpec((BR, F3), lambda i: (i, 0)),
        ] + [pl.BlockSpec((BR, F), lambda i: (i, 0))] * 8,
        out_specs=[
            pl.BlockSpec((BR, F), lambda i: (i, 0)),
            pl.BlockSpec((BR, F3), lambda i: (i, 0)),
        ],
        out_shape=[
            jax.ShapeDtypeStruct((N, F), jnp.float32),
            jax.ShapeDtypeStruct((N, F3), jnp.float32),
        ],
    )(q2, mu_f, p0, p1, *dmu_parts)


def _for_each_chunk(s, fn, nch):
    def body(t, _):
        c = t * NS + s

        @pl.when(c < nch)
        def _():
            fn(pl.multiple_of(c * CH, 8))

        return 0

    lax.fori_loop(0, (nch + NS - 1) // NS, body, 0)


def _zero_fill(zbuf, width):
    def zrow(r, _):
        for g in range(width // L):
            zbuf[r, pl.ds(g * L, L)] = jnp.zeros((L,), jnp.float32)
        return 0

    lax.fori_loop(0, CH, zrow, 0)


_sc_mesh = plsc.VectorSubcoreMesh(
    core_axis_name="c", subcore_axis_name="s", num_cores=NC, num_subcores=NS
)


@functools.partial(
    pl.kernel,
    out_type=[jax.ShapeDtypeStruct((N, F), jnp.float32)] * NC,
    mesh=_sc_mesh,
    scratch_types=[
        pltpu.VMEM((E,), jnp.int32),
        pltpu.VMEM((E,), jnp.int32),
        pltpu.VMEM((E, F), jnp.float32),
        pltpu.VMEM((E, F), jnp.float32),
        pltpu.VMEM((E, F), jnp.float32),
        pltpu.VMEM((CH, F), jnp.float32),
        pltpu.VMEM_SHARED((N + 8, F), jnp.float32),
        pltpu.SemaphoreType.DMA,
    ],
)
def _dq_kernel(xa_hbm, w_hbm, idxi_hbm, idxj_hbm, out0_hbm, out1_hbm,
               idxj_v, idxi_v, rows_v, w_v, contrib_v, zbuf, acc, sem):
    c = lax.axis_index("c")
    s = lax.axis_index("s")

    _zero_fill(zbuf, F)
    _for_each_chunk(s, lambda r0: pltpu.sync_copy(zbuf, acc.at[pl.ds(r0, CH)]),
                    N // CH)
    plsc.subcore_barrier()

    def chunk_body(t, _):
        k = c * NCHUNK2 + t * NS + s
        base = pl.multiple_of(k * E, 8)
        pltpu.sync_copy(idxj_hbm.at[pl.ds(base, E)], idxj_v)
        pltpu.sync_copy(idxi_hbm.at[pl.ds(base, E)], idxi_v)
        cp_x = pltpu.async_copy(xa_hbm.at[idxj_v], rows_v, sem)
        cp_w = pltpu.async_copy(w_hbm.at[pl.ds(base, E), pl.ds(0, F)], w_v, sem)
        cp_x.wait()
        cp_w.wait()

        def e_body(e, _):
            for g in range(F // L):
                sl = pl.ds(g * L, L)
                contrib_v[e, sl] = rows_v[e, sl] * w_v[e, sl]
            return 0

        lax.fori_loop(0, E, e_body, 0)
        pltpu.sync_copy(contrib_v, acc.at[idxi_v], add=True)
        return 0

    lax.fori_loop(0, NCHUNK2 // NS, chunk_body, 0)
    plsc.subcore_barrier()

    def copy_out(r0):
        @pl.when(c == 0)
        def _():
            pltpu.sync_copy(acc.at[pl.ds(r0, CH)], out0_hbm.at[pl.ds(r0, CH)])

        @pl.when(c == 1)
        def _():
            pltpu.sync_copy(acc.at[pl.ds(r0, CH)], out1_hbm.at[pl.ds(r0, CH)])

    _for_each_chunk(s, copy_out, N // CH)


@functools.partial(
    pl.kernel,
    out_type=[jax.ShapeDtypeStruct((N, F), jnp.float32)] * 6,
    mesh=_sc_mesh,
    scratch_types=[
        pltpu.VMEM((EM,), jnp.int32),
        pltpu.VMEM((EM,), jnp.int32),
        pltpu.VMEM((EM, 2 * F), jnp.float32),
        pltpu.VMEM((EM, F3), jnp.float32),
        pltpu.VMEM((EM, 2 * F), jnp.float32),
        pltpu.VMEM((3 * EM * L,), jnp.float32),
        pltpu.VMEM((EM, F), jnp.float32),
        pltpu.VMEM((CH, F), jnp.float32),
        pltpu.VMEM_SHARED((N + 8, F), jnp.float32),
        pltpu.SemaphoreType.DMA,
    ],
)
def _dmu_kernel(xc_hbm, mu_hbm, w_hbm, dir_hbm, idxi_hbm, idxj_hbm,
                o00, o01, o02, o10, o11, o12,
                idxj_v, idxi_v, xc_v, mug_v, w2_v, dir_v,
                contrib_v, zbuf, acc, sem):
    c = lax.axis_index("c")
    s = lax.axis_index("s")
    outs = ((o00, o01, o02), (o10, o11, o12))

    _zero_fill(zbuf, F)

    for d in range(3):
        _for_each_chunk(
            s, lambda r0: pltpu.sync_copy(zbuf, acc.at[pl.ds(r0, CH)]),
            N // CH)
        plsc.subcore_barrier()

        def chunk_body(t, _):
            k = c * NCHM2 + t * NS + s
            base = pl.multiple_of(k * EM, 8)
            pltpu.sync_copy(idxj_hbm.at[pl.ds(base, EM)], idxj_v)
            pltpu.sync_copy(idxi_hbm.at[pl.ds(base, EM)], idxi_v)
            pltpu.sync_copy(
                dir_hbm.at[pl.ds(pl.multiple_of(3 * L * base, 8),
                                 3 * L * EM)],
                dir_v)
            cp_x = pltpu.async_copy(xc_hbm.at[idxj_v], xc_v, sem)
            cp_m = pltpu.async_copy(mu_hbm.at[idxj_v], mug_v, sem)
            cp_w = pltpu.async_copy(
                w_hbm.at[pl.ds(base, EM), pl.ds(F, 2 * F)], w2_v, sem)
            cp_x.wait()
            cp_m.wait()
            cp_w.wait()

            def e_body(e, _):
                off = 3 * L * e
                for g in range(F // L):
                    slg = pl.ds(g * L, L)
                    dmur = xc_v[e, slg] * w2_v[e, slg]
                    dmum = (xc_v[e, pl.ds(F + g * L, L)]
                            * w2_v[e, pl.ds(F + g * L, L)])
                    contrib_v[e, slg] = (
                        dmur * dir_v[pl.ds(off + d * L, L)]
                        + dmum * mug_v[e, pl.ds(d * F + g * L, L)])
                return 0

            lax.fori_loop(0, EM, e_body, 0)
            pltpu.sync_copy(contrib_v, acc.at[idxi_v], add=True)
            return 0

        lax.fori_loop(0, NCHM2 // NS, chunk_body, 0)
        plsc.subcore_barrier()

        def copy_out(r0):
            @pl.when(c == 0)
            def _():
                pltpu.sync_copy(acc.at[pl.ds(r0, CH)],
                                outs[0][d].at[pl.ds(r0, CH)])

            @pl.when(c == 1)
            def _():
                pltpu.sync_copy(acc.at[pl.ds(r0, CH)],
                                outs[1][d].at[pl.ds(r0, CH)])

        _for_each_chunk(s, copy_out, N // CH)
        plsc.subcore_barrier()


def kernel(q, mu, W_ij, dir_ij, pairlist, W1, b1, W2, b2):
    q2 = q.reshape(N, F)
    mu_f = mu.reshape(N, F3)
    xa, xc = _mlp(q2, W1, b1.reshape(1, F), W2, b2.reshape(1, F3))
    idx_i = pairlist[0]
    idx_j = pairlist[1]
    dir_b = jnp.broadcast_to(dir_ij[:, :, None], (P, 3, L)).reshape(-1)

    p0, p1 = _dq_kernel(xa, W_ij, idx_i, idx_j)
    dmu_parts = _dmu_kernel(xc, mu_f, W_ij, dir_b, idx_i, idx_j)

    qo, mo = _epilogue(q2, mu_f, p0, p1, dmu_parts)
    return (qo.reshape(N, 1, F), mo.reshape(N, 3, F))
